# baseline (speedup 1.0000x reference)
import functools

import jax
import jax.numpy as jnp
from jax import lax
from jax.experimental import pallas as pl
from jax.experimental.pallas import tpu as pltpu
from jax.experimental.pallas import tpu_sc as plsc

VOCAB = 49408
MAXPOS = 8192
DIM = 1024
NSPECIAL = 16
LROWS = MAXPOS + NSPECIAL

NC = 2
NS = 16
LANES = 16
NW = NC * NS
RW = MAXPOS // NW
CHUNK = 16
NCHUNK = RW // CHUNK
NBUF = 3
VPR = DIM // LANES


def _sc_body(ids_hbm, tok_hbm, pos_hbm, spec_hbm, out_hbm,
             idx_all, tok_v, pos_v, stage_v, gsems, psems, wsems):
    wid = lax.axis_index("s") * NC + lax.axis_index("c")
    base = wid * RW

    pltpu.sync_copy(ids_hbm.at[pl.ds(NSPECIAL + base, RW)], idx_all)

    def issue(ch, b):
        row0 = base + ch * CHUNK
        g = pltpu.async_copy(tok_hbm.at[idx_all.at[pl.ds(ch * CHUNK, CHUNK)]],
                             tok_v.at[b], gsems[b])
        p = pltpu.async_copy(pos_hbm.at[pl.ds(row0, CHUNK)],
                             pos_v.at[b], psems[b])
        return g, p

    inflight = [None] * NBUF
    writes = [None] * NBUF
    for pre in range(NBUF - 1):
        inflight[pre] = issue(pre, pre)

    for ch in range(NCHUNK):
        b = ch % NBUF
        ahead = ch + NBUF - 1
        if ahead < NCHUNK:
            ab = ahead % NBUF
            if writes[ab] is not None:
                writes[ab].wait()
                writes[ab] = None
            inflight[ab] = issue(ahead, ab)
        g, p = inflight[b]
        g.wait()
        p.wait()

        def add_row(r, carry, _b=b):
            for k in range(VPR):
                sl = pl.ds(k * LANES, LANES)
                plsc.addupdate(tok_v.at[_b, r, sl], pos_v[_b, r, sl])
            return carry

        if ch == 0:
            @pl.when(wid == 0)
            def _():
                pltpu.sync_copy(spec_hbm, stage_v)
                for k in range(VPR):
                    sl = pl.ds(k * LANES, LANES)
                    stage_v[0, sl] = stage_v[NSPECIAL - 1, sl]

                def add_shift(r, carry):
                    for k in range(VPR):
                        sl = pl.ds(k * LANES, LANES)
                        stage_v[r, sl] = tok_v[b, r, sl] + pos_v[b, r, sl]
                    return carry
                lax.fori_loop(1, CHUNK, add_shift, 0)
                pltpu.sync_copy(stage_v,
                                out_hbm.at[pl.ds(NSPECIAL, NSPECIAL)])

                pltpu.sync_copy(spec_hbm, stage_v)

                def shift_down(i, carry):
                    r = NSPECIAL - 2 - i
                    for k in range(VPR):
                        sl = pl.ds(k * LANES, LANES)
                        stage_v[r + 1, sl] = stage_v[r, sl]
                    return carry
                lax.fori_loop(0, NSPECIAL - 1, shift_down, 0)
                for k in range(VPR):
                    sl = pl.ds(k * LANES, LANES)
                    stage_v[0, sl] = tok_v[b, 0, sl] + pos_v[b, 0, sl]
                pltpu.sync_copy(stage_v, out_hbm.at[pl.ds(0, NSPECIAL)])

            @pl.when(wid != 0)
            def _():
                lax.fori_loop(0, CHUNK, add_row, 0)
                pltpu.sync_copy(tok_v.at[b],
                                out_hbm.at[pl.ds(base + NSPECIAL, CHUNK)])
        else:
            lax.fori_loop(0, CHUNK, add_row, 0)
            row0 = base + ch * CHUNK
            writes[b] = pltpu.async_copy(
                tok_v.at[b], out_hbm.at[pl.ds(row0 + NSPECIAL, CHUNK)],
                wsems[b])

    for w in writes:
        if w is not None:
            w.wait()


_sc_kernel = functools.partial(
    pl.kernel,
    out_type=jax.ShapeDtypeStruct((LROWS, DIM), jnp.float32),
    mesh=plsc.VectorSubcoreMesh(core_axis_name="c", subcore_axis_name="s"),
    scratch_types=[
        pltpu.VMEM((RW,), jnp.int32),
        pltpu.VMEM((NBUF, CHUNK, DIM), jnp.float32),
        pltpu.VMEM((NBUF, CHUNK, DIM), jnp.float32),
        pltpu.VMEM((NSPECIAL, DIM), jnp.float32),
        [pltpu.SemaphoreType.DMA] * NBUF,
        [pltpu.SemaphoreType.DMA] * NBUF,
        [pltpu.SemaphoreType.DMA] * NBUF,
    ],
)(_sc_body)


def kernel(input_ids, token_table, pos_table, special_token_embedding):
    ids = input_ids.reshape(LROWS)
    spec = special_token_embedding.reshape(NSPECIAL, DIM)
    out = _sc_kernel(ids, token_table, pos_table, spec)
    return out.reshape(1, LROWS, DIM)
